# Initial kernel scaffold; baseline (speedup 1.0000x reference)
import jax
import jax.numpy as jnp
from jax.experimental import pallas as pl


def kernel(span_width, span_width_embeddings):
    raise NotImplementedError("write your pallas kernel here")


# speedup vs baseline: 2.9472x; 2.9472x over previous
import functools

import jax
import jax.numpy as jnp
from jax import lax
from jax.experimental import pallas as pl
from jax.experimental.pallas import tpu as pltpu
from jax.experimental.pallas import tpu_sc as plsc

_BATCH = 16384
_N_SPANS = 20
_D = 128
_B_TOTAL = _BATCH * _N_SPANS
_NUM_CORES = 2
_NUM_SUBCORES = 16
_NW = _NUM_CORES * _NUM_SUBCORES
_B_PER_W = _B_TOTAL // _NW
_CHUNK = 256
_N_CHUNKS = _B_PER_W // _CHUNK
_LANES = 16


def _sc_gather(table_hbm, idx_hbm, out_hbm, idx_v, rows_v, sem):
    wid = lax.axis_index("s") * _NUM_CORES + lax.axis_index("c")
    base = wid * _B_PER_W

    pltpu.sync_copy(idx_hbm.at[pl.ds(base, _B_PER_W)], idx_v)

    @pl.loop(0, _B_PER_W // _LANES)
    def _sub1(i):
        sl = pl.ds(i * _LANES, _LANES)
        idx_v[sl] = idx_v[sl] - 1

    @pl.loop(0, _N_CHUNKS)
    def _chunk(c):
        pltpu.async_copy(
            table_hbm.at[idx_v.at[pl.ds(c * _CHUNK, _CHUNK)]], rows_v, sem
        ).wait()
        pltpu.sync_copy(rows_v, out_hbm.at[pl.ds(base + c * _CHUNK, _CHUNK)])


def kernel(span_width, span_width_embeddings):
    idx = span_width.reshape(_B_TOTAL)
    mesh = plsc.VectorSubcoreMesh(
        core_axis_name="c",
        subcore_axis_name="s",
        num_cores=_NUM_CORES,
        num_subcores=_NUM_SUBCORES,
    )
    run = functools.partial(
        pl.kernel,
        mesh=mesh,
        out_type=jax.ShapeDtypeStruct((_B_TOTAL, _D), jnp.float32),
        scratch_types=[
            pltpu.VMEM((_B_PER_W,), jnp.int32),
            pltpu.VMEM((_CHUNK, _D), jnp.float32),
            pltpu.SemaphoreType.DMA,
        ],
    )(_sc_gather)
    out = run(span_width_embeddings, idx)
    return out.reshape(_BATCH, _N_SPANS, _D)


# speedup vs baseline: 2.9624x; 1.0052x over previous
import functools

import jax
import jax.numpy as jnp
from jax import lax
from jax.experimental import pallas as pl
from jax.experimental.pallas import tpu as pltpu
from jax.experimental.pallas import tpu_sc as plsc

_BATCH = 16384
_N_SPANS = 20
_D = 128
_B_TOTAL = _BATCH * _N_SPANS
_NUM_CORES = 2
_NUM_SUBCORES = 16
_NW = _NUM_CORES * _NUM_SUBCORES
_B_PER_W = _B_TOTAL // _NW
_CHUNK = 256
_N_CHUNKS = _B_PER_W // _CHUNK
_NBUF = 2
assert _N_CHUNKS % _NBUF == 0
_LANES = 16


def _sc_gather(table_hbm, idx_hbm, out_hbm, idx_v, *bufs_and_sems):
    bufs = bufs_and_sems[:_NBUF]
    gsems = bufs_and_sems[_NBUF:2 * _NBUF]
    osems = bufs_and_sems[2 * _NBUF:3 * _NBUF]

    wid = lax.axis_index("s") * _NUM_CORES + lax.axis_index("c")
    base = wid * _B_PER_W

    pltpu.sync_copy(idx_hbm.at[pl.ds(base, _B_PER_W)], idx_v)

    @pl.loop(0, _B_PER_W // _LANES)
    def _sub1(i):
        sl = pl.ds(i * _LANES, _LANES)
        idx_v[sl] = idx_v[sl] - 1

    def start_gather(c, b):
        pltpu.async_copy(
            table_hbm.at[idx_v.at[pl.ds(c * _CHUNK, _CHUNK)]], bufs[b], gsems[b]
        )

    def wait_gather(c, b):
        pltpu.make_async_copy(
            table_hbm.at[idx_v.at[pl.ds(c * _CHUNK, _CHUNK)]], bufs[b], gsems[b]
        ).wait()

    def start_out(c, b):
        pltpu.async_copy(
            bufs[b], out_hbm.at[pl.ds(base + c * _CHUNK, _CHUNK)], osems[b]
        )

    def wait_out(c, b):
        pltpu.make_async_copy(
            bufs[b], out_hbm.at[pl.ds(base + c * _CHUNK, _CHUNK)], osems[b]
        ).wait()

    for b in range(_NBUF):
        start_gather(b, b)

    @pl.loop(0, _N_CHUNKS - _NBUF, step=_NBUF)
    def _chunk(c0):
        for b in range(_NBUF):
            c = c0 + b
            wait_gather(c, b)
            start_out(c, b)
            wait_out(c, b)
            start_gather(c + _NBUF, b)

    for b in range(_NBUF):
        c = _N_CHUNKS - _NBUF + b
        wait_gather(c, b)
        start_out(c, b)
        wait_out(c, b)


def kernel(span_width, span_width_embeddings):
    idx = span_width.reshape(_B_TOTAL)
    mesh = plsc.VectorSubcoreMesh(
        core_axis_name="c",
        subcore_axis_name="s",
        num_cores=_NUM_CORES,
        num_subcores=_NUM_SUBCORES,
    )
    run = functools.partial(
        pl.kernel,
        mesh=mesh,
        out_type=jax.ShapeDtypeStruct((_B_TOTAL, _D), jnp.float32),
        scratch_types=(
            [pltpu.VMEM((_B_PER_W,), jnp.int32)]
            + [pltpu.VMEM((_CHUNK, _D), jnp.float32) for _ in range(_NBUF)]
            + [pltpu.SemaphoreType.DMA for _ in range(2 * _NBUF)]
        ),
    )(_sc_gather)
    out = run(span_width_embeddings, idx)
    return out.reshape(_BATCH, _N_SPANS, _D)


# speedup vs baseline: 4.6001x; 1.5528x over previous
import functools

import jax
import jax.numpy as jnp
from jax import lax
from jax.experimental import pallas as pl
from jax.experimental.pallas import tpu as pltpu
from jax.experimental.pallas import tpu_sc as plsc

_BATCH = 16384
_N_SPANS = 20
_D = 128
_B_TOTAL = _BATCH * _N_SPANS
_NUM_CORES = 2
_NUM_SUBCORES = 16
_NW = _NUM_CORES * _NUM_SUBCORES
_B_PER_W = _BATCH // _NW
_ROWS_PER_W = _B_PER_W * _N_SPANS
_CB = 16
_CHUNK = _CB * _N_SPANS
_N_CHUNKS = _B_PER_W // _CB
_NBUF = 2
assert _N_CHUNKS % _NBUF == 0
_LANES = 16


def _sc_gather(table_hbm, idx_hbm, out_hbm, idx_v, *bufs_and_sems):
    bufs = bufs_and_sems[:_NBUF]
    gsems = bufs_and_sems[_NBUF:2 * _NBUF]
    osems = bufs_and_sems[2 * _NBUF:3 * _NBUF]

    wid = lax.axis_index("s") * _NUM_CORES + lax.axis_index("c")
    row_base = wid * _ROWS_PER_W
    b_base = wid * _B_PER_W

    pltpu.sync_copy(idx_hbm.at[pl.ds(row_base, _ROWS_PER_W)], idx_v)

    @pl.loop(0, _ROWS_PER_W // _LANES)
    def _sub1(i):
        sl = pl.ds(i * _LANES, _LANES)
        idx_v[sl] = idx_v[sl] - 1

    def start_gather(c, b):
        pltpu.async_copy(
            table_hbm.at[idx_v.at[pl.ds(c * _CHUNK, _CHUNK)]], bufs[b], gsems[b]
        )

    def wait_gather(c, b):
        pltpu.make_async_copy(
            table_hbm.at[idx_v.at[pl.ds(c * _CHUNK, _CHUNK)]], bufs[b], gsems[b]
        ).wait()

    def start_out(c, b):
        for j in range(_CB):
            pltpu.async_copy(
                bufs[b].at[pl.ds(j * _N_SPANS, _N_SPANS)],
                out_hbm.at[b_base + c * _CB + j],
                osems[b],
            )

    def wait_out(c, b):
        for j in range(_CB):
            pltpu.make_async_copy(
                bufs[b].at[pl.ds(j * _N_SPANS, _N_SPANS)],
                out_hbm.at[b_base + c * _CB + j],
                osems[b],
            ).wait()

    for b in range(_NBUF):
        start_gather(b, b)

    @pl.loop(0, _N_CHUNKS - _NBUF, step=_NBUF)
    def _chunk(c0):
        for b in range(_NBUF):
            c = c0 + b
            wait_gather(c, b)
            start_out(c, b)
            wait_out(c, b)
            start_gather(c + _NBUF, b)

    for b in range(_NBUF):
        c = _N_CHUNKS - _NBUF + b
        wait_gather(c, b)
        start_out(c, b)
        wait_out(c, b)


def kernel(span_width, span_width_embeddings):
    idx = span_width.reshape(_B_TOTAL)
    mesh = plsc.VectorSubcoreMesh(
        core_axis_name="c",
        subcore_axis_name="s",
        num_cores=_NUM_CORES,
        num_subcores=_NUM_SUBCORES,
    )
    run = functools.partial(
        pl.kernel,
        mesh=mesh,
        out_type=jax.ShapeDtypeStruct((_BATCH, _N_SPANS, _D), jnp.float32),
        scratch_types=(
            [pltpu.VMEM((_ROWS_PER_W,), jnp.int32)]
            + [pltpu.VMEM((_CHUNK, _D), jnp.float32) for _ in range(_NBUF)]
            + [pltpu.SemaphoreType.DMA for _ in range(2 * _NBUF)]
        ),
        compiler_params=pltpu.CompilerParams(use_tc_tiling_on_sc=True),
    )(_sc_gather)
    return run(span_width_embeddings, idx)


# speedup vs baseline: 7.4776x; 1.6255x over previous
import functools

import jax
import jax.numpy as jnp
from jax import lax
from jax.experimental import pallas as pl
from jax.experimental.pallas import tpu as pltpu
from jax.experimental.pallas import tpu_sc as plsc

_BATCH = 16384
_N_SPANS = 20
_D = 128
_B_TOTAL = _BATCH * _N_SPANS
_NUM_CORES = 2
_NUM_SUBCORES = 16
_NW = _NUM_CORES * _NUM_SUBCORES
_B_PER_W = _B_TOTAL // _NW
_CHUNK = 256
_N_CHUNKS = _B_PER_W // _CHUNK
_NBUF = 2
assert _N_CHUNKS % _NBUF == 0
_LANES = 16


def _sc_gather(table_hbm, idx_hbm, out_hbm, idx_v, *bufs_and_sems):
    bufs = bufs_and_sems[:_NBUF]
    gsems = bufs_and_sems[_NBUF:2 * _NBUF]
    osems = bufs_and_sems[2 * _NBUF:3 * _NBUF]

    wid = lax.axis_index("s") * _NUM_CORES + lax.axis_index("c")
    base = wid * _B_PER_W

    pltpu.sync_copy(idx_hbm.at[pl.ds(base, _B_PER_W)], idx_v)

    @pl.loop(0, _B_PER_W // _LANES)
    def _sub1(i):
        sl = pl.ds(i * _LANES, _LANES)
        idx_v[sl] = idx_v[sl] - 1

    def start_gather(c, b):
        pltpu.async_copy(
            table_hbm.at[idx_v.at[pl.ds(c * _CHUNK, _CHUNK)]], bufs[b], gsems[b]
        )

    def wait_gather(c, b):
        pltpu.make_async_copy(
            table_hbm.at[idx_v.at[pl.ds(c * _CHUNK, _CHUNK)]], bufs[b], gsems[b]
        ).wait()

    def start_out(c, b):
        pltpu.async_copy(
            bufs[b], out_hbm.at[pl.ds(base + c * _CHUNK, _CHUNK)], osems[b]
        )

    def wait_out(c, b):
        pltpu.make_async_copy(
            bufs[b], out_hbm.at[pl.ds(base + c * _CHUNK, _CHUNK)], osems[b]
        ).wait()

    for b in range(_NBUF):
        start_gather(b, b)

    @pl.loop(0, _N_CHUNKS - _NBUF, step=_NBUF)
    def _chunk(c0):
        for b in range(_NBUF):
            c = c0 + b
            wait_gather(c, b)
            start_out(c, b)
            wait_out(c, b)
            start_gather(c + _NBUF, b)

    for b in range(_NBUF):
        c = _N_CHUNKS - _NBUF + b
        wait_gather(c, b)
        start_out(c, b)
        wait_out(c, b)


def kernel(span_width, span_width_embeddings):
    idx = span_width.T.reshape(_B_TOTAL)
    mesh = plsc.VectorSubcoreMesh(
        core_axis_name="c",
        subcore_axis_name="s",
        num_cores=_NUM_CORES,
        num_subcores=_NUM_SUBCORES,
    )
    run = functools.partial(
        pl.kernel,
        mesh=mesh,
        out_type=jax.ShapeDtypeStruct((_B_TOTAL, _D), jnp.float32),
        scratch_types=(
            [pltpu.VMEM((_B_PER_W,), jnp.int32)]
            + [pltpu.VMEM((_CHUNK, _D), jnp.float32) for _ in range(_NBUF)]
            + [pltpu.SemaphoreType.DMA for _ in range(2 * _NBUF)]
        ),
        compiler_params=pltpu.CompilerParams(use_tc_tiling_on_sc=True),
    )(_sc_gather)
    out = run(span_width_embeddings, idx)
    return out.reshape(_N_SPANS, _BATCH, _D).transpose(1, 0, 2)


# speedup vs baseline: 10.1445x; 1.3567x over previous
import functools

import jax
import jax.numpy as jnp
from jax import lax
from jax.experimental import pallas as pl
from jax.experimental.pallas import tpu as pltpu
from jax.experimental.pallas import tpu_sc as plsc

_BATCH = 16384
_N_SPANS = 20
_D = 128
_B_TOTAL = _BATCH * _N_SPANS
_NUM_CORES = 2
_NUM_SUBCORES = 16
_NW = _NUM_CORES * _NUM_SUBCORES
_B_PER_W = _B_TOTAL // _NW
_CHUNK = 256
_N_CHUNKS = _B_PER_W // _CHUNK
_NBUF = 2
assert _N_CHUNKS % _NBUF == 0
_LANES = 16
_TABLE_ROWS = 1000
_REPS = 8


def _sc_gather(table_hbm, idx_hbm, out_hbm, idx_v, *bufs_and_sems):
    bufs = bufs_and_sems[:_NBUF]
    gsems = bufs_and_sems[_NBUF:2 * _NBUF]
    osems = bufs_and_sems[2 * _NBUF:3 * _NBUF]

    wid = lax.axis_index("s") * _NUM_CORES + lax.axis_index("c")
    base = wid * _B_PER_W

    pltpu.sync_copy(idx_hbm.at[pl.ds(base, _B_PER_W)], idx_v)

    rep_off = (wid % _REPS) * _TABLE_ROWS - 1

    @pl.loop(0, _B_PER_W // _LANES)
    def _sub1(i):
        sl = pl.ds(i * _LANES, _LANES)
        idx_v[sl] = idx_v[sl] + rep_off

    def start_gather(c, b):
        pltpu.async_copy(
            table_hbm.at[idx_v.at[pl.ds(c * _CHUNK, _CHUNK)]], bufs[b], gsems[b]
        )

    def wait_gather(c, b):
        pltpu.make_async_copy(
            table_hbm.at[idx_v.at[pl.ds(c * _CHUNK, _CHUNK)]], bufs[b], gsems[b]
        ).wait()

    def start_out(c, b):
        pltpu.async_copy(
            bufs[b], out_hbm.at[pl.ds(base + c * _CHUNK, _CHUNK)], osems[b]
        )

    def wait_out(c, b):
        pltpu.make_async_copy(
            bufs[b], out_hbm.at[pl.ds(base + c * _CHUNK, _CHUNK)], osems[b]
        ).wait()

    for b in range(_NBUF):
        start_gather(b, b)

    @pl.loop(0, _N_CHUNKS - _NBUF, step=_NBUF)
    def _chunk(c0):
        for b in range(_NBUF):
            c = c0 + b
            wait_gather(c, b)
            start_out(c, b)
            wait_out(c, b)
            start_gather(c + _NBUF, b)

    for b in range(_NBUF):
        c = _N_CHUNKS - _NBUF + b
        wait_gather(c, b)
        start_out(c, b)
        wait_out(c, b)


def kernel(span_width, span_width_embeddings):
    idx = span_width.T.reshape(_B_TOTAL)
    table_rep = jnp.tile(span_width_embeddings, (_REPS, 1))
    mesh = plsc.VectorSubcoreMesh(
        core_axis_name="c",
        subcore_axis_name="s",
        num_cores=_NUM_CORES,
        num_subcores=_NUM_SUBCORES,
    )
    run = functools.partial(
        pl.kernel,
        mesh=mesh,
        out_type=jax.ShapeDtypeStruct((_B_TOTAL, _D), jnp.float32),
        scratch_types=(
            [pltpu.VMEM((_B_PER_W,), jnp.int32)]
            + [pltpu.VMEM((_CHUNK, _D), jnp.float32) for _ in range(_NBUF)]
            + [pltpu.SemaphoreType.DMA for _ in range(2 * _NBUF)]
        ),
        compiler_params=pltpu.CompilerParams(use_tc_tiling_on_sc=True),
    )(_sc_gather)
    out = run(table_rep, idx)
    return out.reshape(_N_SPANS, _BATCH, _D).transpose(1, 0, 2)


# speedup vs baseline: 10.3046x; 1.0158x over previous
import functools

import jax
import jax.numpy as jnp
from jax import lax
from jax.experimental import pallas as pl
from jax.experimental.pallas import tpu as pltpu
from jax.experimental.pallas import tpu_sc as plsc

_BATCH = 16384
_N_SPANS = 20
_D = 128
_B_TOTAL = _BATCH * _N_SPANS
_NUM_CORES = 2
_NUM_SUBCORES = 16
_NW = _NUM_CORES * _NUM_SUBCORES
_B_PER_W = _B_TOTAL // _NW
_CHUNK = 256
_N_CHUNKS = _B_PER_W // _CHUNK
_NBUF = 2
assert _N_CHUNKS % _NBUF == 0
_LANES = 16
_TABLE_ROWS = 1000
_REPS = 32


def _sc_gather(table_hbm, idx_hbm, out_hbm, idx_v, *bufs_and_sems):
    bufs = bufs_and_sems[:_NBUF]
    gsems = bufs_and_sems[_NBUF:2 * _NBUF]
    osems = bufs_and_sems[2 * _NBUF:3 * _NBUF]

    wid = lax.axis_index("s") * _NUM_CORES + lax.axis_index("c")
    base = wid * _B_PER_W

    pltpu.sync_copy(idx_hbm.at[pl.ds(base, _B_PER_W)], idx_v)

    rep_off = (wid % _REPS) * _TABLE_ROWS - 1

    @pl.loop(0, _B_PER_W // _LANES)
    def _sub1(i):
        sl = pl.ds(i * _LANES, _LANES)
        idx_v[sl] = idx_v[sl] + rep_off

    def start_gather(c, b):
        pltpu.async_copy(
            table_hbm.at[idx_v.at[pl.ds(c * _CHUNK, _CHUNK)]], bufs[b], gsems[b]
        )

    def wait_gather(c, b):
        pltpu.make_async_copy(
            table_hbm.at[idx_v.at[pl.ds(c * _CHUNK, _CHUNK)]], bufs[b], gsems[b]
        ).wait()

    def start_out(c, b):
        pltpu.async_copy(
            bufs[b], out_hbm.at[pl.ds(base + c * _CHUNK, _CHUNK)], osems[b]
        )

    def wait_out(c, b):
        pltpu.make_async_copy(
            bufs[b], out_hbm.at[pl.ds(base + c * _CHUNK, _CHUNK)], osems[b]
        ).wait()

    for b in range(_NBUF):
        start_gather(b, b)

    @pl.loop(0, _N_CHUNKS - _NBUF, step=_NBUF)
    def _chunk(c0):
        for b in range(_NBUF):
            c = c0 + b
            wait_gather(c, b)
            start_out(c, b)
            wait_out(c, b)
            start_gather(c + _NBUF, b)

    for b in range(_NBUF):
        c = _N_CHUNKS - _NBUF + b
        wait_gather(c, b)
        start_out(c, b)
        wait_out(c, b)


def kernel(span_width, span_width_embeddings):
    idx = span_width.T.reshape(_B_TOTAL)
    table_rep = jnp.tile(span_width_embeddings, (_REPS, 1))
    mesh = plsc.VectorSubcoreMesh(
        core_axis_name="c",
        subcore_axis_name="s",
        num_cores=_NUM_CORES,
        num_subcores=_NUM_SUBCORES,
    )
    run = functools.partial(
        pl.kernel,
        mesh=mesh,
        out_type=jax.ShapeDtypeStruct((_B_TOTAL, _D), jnp.float32),
        scratch_types=(
            [pltpu.VMEM((_B_PER_W,), jnp.int32)]
            + [pltpu.VMEM((_CHUNK, _D), jnp.float32) for _ in range(_NBUF)]
            + [pltpu.SemaphoreType.DMA for _ in range(2 * _NBUF)]
        ),
        compiler_params=pltpu.CompilerParams(use_tc_tiling_on_sc=True),
    )(_sc_gather)
    out = run(table_rep, idx)
    return out.reshape(_N_SPANS, _BATCH, _D).transpose(1, 0, 2)


# speedup vs baseline: 10.4657x; 1.0156x over previous
import functools

import jax
import jax.numpy as jnp
from jax import lax
from jax.experimental import pallas as pl
from jax.experimental.pallas import tpu as pltpu
from jax.experimental.pallas import tpu_sc as plsc

_BATCH = 16384
_N_SPANS = 20
_D = 128
_B_TOTAL = _BATCH * _N_SPANS
_NUM_CORES = 2
_NUM_SUBCORES = 16
_NW = _NUM_CORES * _NUM_SUBCORES
_B_PER_W = _B_TOTAL // _NW
_CHUNK = 160
_N_CHUNKS = _B_PER_W // _CHUNK
_NBUF = 4
assert _N_CHUNKS % _NBUF == 0
_LANES = 16
_TABLE_ROWS = 1000
_REPS = 32


def _sc_gather(table_hbm, idx_hbm, out_hbm, idx_v, *bufs_and_sems):
    bufs = bufs_and_sems[:_NBUF]
    gsems = bufs_and_sems[_NBUF:2 * _NBUF]
    osems = bufs_and_sems[2 * _NBUF:3 * _NBUF]

    wid = lax.axis_index("s") * _NUM_CORES + lax.axis_index("c")
    base = wid * _B_PER_W

    pltpu.sync_copy(idx_hbm.at[pl.ds(base, _B_PER_W)], idx_v)

    rep_off = (wid % _REPS) * _TABLE_ROWS - 1

    @pl.loop(0, _B_PER_W // _LANES)
    def _sub1(i):
        sl = pl.ds(i * _LANES, _LANES)
        idx_v[sl] = idx_v[sl] + rep_off

    def start_gather(c, b):
        pltpu.async_copy(
            table_hbm.at[idx_v.at[pl.ds(c * _CHUNK, _CHUNK)]], bufs[b], gsems[b]
        )

    def wait_gather(c, b):
        pltpu.make_async_copy(
            table_hbm.at[idx_v.at[pl.ds(c * _CHUNK, _CHUNK)]], bufs[b], gsems[b]
        ).wait()

    def start_out(c, b):
        pltpu.async_copy(
            bufs[b], out_hbm.at[pl.ds(base + c * _CHUNK, _CHUNK)], osems[b]
        )

    def wait_out(c, b):
        pltpu.make_async_copy(
            bufs[b], out_hbm.at[pl.ds(base + c * _CHUNK, _CHUNK)], osems[b]
        ).wait()

    for b in range(_NBUF):
        start_gather(b, b)

    @pl.loop(0, _N_CHUNKS - _NBUF, step=_NBUF)
    def _chunk(c0):
        for b in range(_NBUF):
            c = c0 + b
            wait_gather(c, b)
            start_out(c, b)
            wait_out(c, b)
            start_gather(c + _NBUF, b)

    for b in range(_NBUF):
        c = _N_CHUNKS - _NBUF + b
        wait_gather(c, b)
        start_out(c, b)
        wait_out(c, b)


def kernel(span_width, span_width_embeddings):
    idx = span_width.T.reshape(_B_TOTAL)
    table_rep = jnp.tile(span_width_embeddings, (_REPS, 1))
    mesh = plsc.VectorSubcoreMesh(
        core_axis_name="c",
        subcore_axis_name="s",
        num_cores=_NUM_CORES,
        num_subcores=_NUM_SUBCORES,
    )
    run = functools.partial(
        pl.kernel,
        mesh=mesh,
        out_type=jax.ShapeDtypeStruct((_B_TOTAL, _D), jnp.float32),
        scratch_types=(
            [pltpu.VMEM((_B_PER_W,), jnp.int32)]
            + [pltpu.VMEM((_CHUNK, _D), jnp.float32) for _ in range(_NBUF)]
            + [pltpu.SemaphoreType.DMA for _ in range(2 * _NBUF)]
        ),
        compiler_params=pltpu.CompilerParams(use_tc_tiling_on_sc=True),
    )(_sc_gather)
    out = run(table_rep, idx)
    return out.reshape(_N_SPANS, _BATCH, _D).transpose(1, 0, 2)


# speedup vs baseline: 10.6953x; 1.0219x over previous
import functools

import jax
import jax.numpy as jnp
from jax import lax
from jax.experimental import pallas as pl
from jax.experimental.pallas import tpu as pltpu
from jax.experimental.pallas import tpu_sc as plsc

_BATCH = 16384
_N_SPANS = 20
_D = 128
_B_TOTAL = _BATCH * _N_SPANS
_NUM_CORES = 2
_NUM_SUBCORES = 16
_NW = _NUM_CORES * _NUM_SUBCORES
_B_PER_W = _B_TOTAL // _NW
_CHUNK = 160
_N_CHUNKS = _B_PER_W // _CHUNK
_NBUF = 4
assert _N_CHUNKS % _NBUF == 0
_LANES = 16
_TABLE_ROWS = 1000
_REPS = 16


def _sc_gather(table_hbm, idx_hbm, out_hbm, idx_v, *bufs_and_sems):
    bufs = bufs_and_sems[:_NBUF]
    gsems = bufs_and_sems[_NBUF:2 * _NBUF]
    osems = bufs_and_sems[2 * _NBUF:3 * _NBUF]

    wid = lax.axis_index("s") * _NUM_CORES + lax.axis_index("c")
    base = wid * _B_PER_W

    pltpu.sync_copy(idx_hbm.at[pl.ds(base, _B_PER_W)], idx_v)

    def start_gather(c, b):
        pltpu.async_copy(
            table_hbm.at[idx_v.at[pl.ds(c * _CHUNK, _CHUNK)]], bufs[b], gsems[b]
        )

    def wait_gather(c, b):
        pltpu.make_async_copy(
            table_hbm.at[idx_v.at[pl.ds(c * _CHUNK, _CHUNK)]], bufs[b], gsems[b]
        ).wait()

    def start_out(c, b):
        pltpu.async_copy(
            bufs[b], out_hbm.at[pl.ds(base + c * _CHUNK, _CHUNK)], osems[b]
        )

    def wait_out(c, b):
        pltpu.make_async_copy(
            bufs[b], out_hbm.at[pl.ds(base + c * _CHUNK, _CHUNK)], osems[b]
        ).wait()

    for b in range(_NBUF):
        start_gather(b, b)

    @pl.loop(0, _N_CHUNKS - _NBUF, step=_NBUF)
    def _chunk(c0):
        for b in range(_NBUF):
            c = c0 + b
            wait_gather(c, b)
            start_out(c, b)
            wait_out(c, b)
            start_gather(c + _NBUF, b)

    for b in range(_NBUF):
        c = _N_CHUNKS - _NBUF + b
        wait_gather(c, b)
        start_out(c, b)
        wait_out(c, b)


def kernel(span_width, span_width_embeddings):
    idx = span_width.T.reshape(_B_TOTAL)
    rep = (jnp.arange(_B_TOTAL, dtype=jnp.int32) // _B_PER_W) % _REPS
    idx = idx - 1 + rep * _TABLE_ROWS
    table_rep = jnp.tile(span_width_embeddings, (_REPS, 1))
    mesh = plsc.VectorSubcoreMesh(
        core_axis_name="c",
        subcore_axis_name="s",
        num_cores=_NUM_CORES,
        num_subcores=_NUM_SUBCORES,
    )
    run = functools.partial(
        pl.kernel,
        mesh=mesh,
        out_type=jax.ShapeDtypeStruct((_B_TOTAL, _D), jnp.float32),
        scratch_types=(
            [pltpu.VMEM((_B_PER_W,), jnp.int32)]
            + [pltpu.VMEM((_CHUNK, _D), jnp.float32) for _ in range(_NBUF)]
            + [pltpu.SemaphoreType.DMA for _ in range(2 * _NBUF)]
        ),
        compiler_params=pltpu.CompilerParams(use_tc_tiling_on_sc=True),
    )(_sc_gather)
    out = run(table_rep, idx)
    return out.reshape(_N_SPANS, _BATCH, _D).transpose(1, 0, 2)
